# baseline (speedup 1.0000x reference)
import functools

import jax
import jax.numpy as jnp
from jax import lax
from jax.experimental import pallas as pl
from jax.experimental.pallas import tpu as pltpu
from jax.experimental.pallas import tpu_sc as plsc

_B, _L, _P, _D = 128, 200, 2048, 128
_BL = _B * _L

_NC, _NS = 2, 16
_NW = _NC * _NS

_PER_W = _BL // _NW
_CHUNK = 80
_NCHUNK = _PER_W // _CHUNK


@functools.cache
def _make_gather_rows():
    @functools.partial(
        pl.kernel,
        out_type=jax.ShapeDtypeStruct((_BL, _D), jnp.float32),
        mesh=plsc.VectorSubcoreMesh(core_axis_name="c", subcore_axis_name="s",
                                    num_cores=_NC, num_subcores=_NS),
        scratch_types=[
            pltpu.VMEM((_NCHUNK, _CHUNK), jnp.int32),
            pltpu.VMEM((_PER_W, _D), jnp.float32),
            pltpu.SemaphoreType.DMA,
        ],
    )
    def _gather_rows(idx_hbm, table_hbm, out_hbm, idx_v, rows_v, sem):
        wid = lax.axis_index("s") * _NC + lax.axis_index("c")
        base = wid * _PER_W
        pltpu.sync_copy(idx_hbm.at[wid], idx_v)
        copies = []
        for j in range(_NCHUNK):
            copies.append(pltpu.async_copy(
                table_hbm.at[idx_v.at[j]],
                rows_v.at[pl.ds(j * _CHUNK, _CHUNK)], sem))
        for c in copies:
            c.wait()
        pltpu.sync_copy(rows_v, out_hbm.at[pl.ds(base, _PER_W)])

    return _gather_rows


_BB = 16


def _enc_body(tok_ref, w_ref, b_ref, out_ref):
    t = tok_ref[...].reshape(_BB * _L, _D)
    y = jnp.tanh(jnp.dot(t, w_ref[...], preferred_element_type=jnp.float32)
                 + b_ref[...])
    m = jnp.mean(y.reshape(_BB, _L, _D), axis=1)
    n = jnp.sqrt(jnp.sum(m * m, axis=1, keepdims=True))
    out_ref[...] = m / jnp.clip(n, 1e-12, None)


_SCB = 64
_T = _B - _SCB
_BPW = _SCB // _NW
_RPW = _BPW * _P
_RCH = 256
_NRCH = _RPW // _RCH
_CPB = _P // _RCH

_DN = (((1,), (1,)), ((), ()))
_NB = 8


def _scores_body(cand_ref, state_ref, out_ref):
    s = state_ref[0].astype(jnp.bfloat16)
    ones = jnp.ones((1, _D), jnp.bfloat16)
    parts = []
    for b in range(_NB):
        c = cand_ref[b * _P:(b + 1) * _P]
        cb = c.astype(jnp.bfloat16)
        dot = lax.dot_general(s[b:b + 1], cb, _DN,
                              preferred_element_type=jnp.float32)
        sq = lax.dot_general(ones, cb * cb, _DN,
                             preferred_element_type=jnp.float32)
        parts.append(dot / jnp.clip(jnp.sqrt(sq), 1e-12, None))
    out_ref[0] = jnp.concatenate(parts, axis=1)


@functools.cache
def _make_sc_scores():
    @functools.partial(
        pl.kernel,
        out_type=(
            jax.ShapeDtypeStruct((_SCB * _P * 16,), jnp.float32),
            jax.ShapeDtypeStruct((_SCB * _P * 16,), jnp.float32),
        ),
        mesh=plsc.VectorSubcoreMesh(core_axis_name="c", subcore_axis_name="s",
                                    num_cores=_NC, num_subcores=_NS),
        scratch_types=[
            pltpu.VMEM((_RCH, _D), jnp.float32),
            pltpu.VMEM((_RCH, _D), jnp.float32),
            pltpu.VMEM((_RCH * 16,), jnp.float32),
            pltpu.VMEM((_RCH * 16,), jnp.float32),
            pltpu.VMEM((_RCH * 16,), jnp.float32),
            pltpu.VMEM((_RCH * 16,), jnp.float32),
            pltpu.VMEM((_BPW * _D,), jnp.float32),
            pltpu.SemaphoreType.DMA,
            pltpu.SemaphoreType.DMA,
            pltpu.SemaphoreType.DMA,
            pltpu.SemaphoreType.DMA,
            pltpu.SemaphoreType.DMA,
            pltpu.SemaphoreType.DMA,
        ],
    )
    def _sc_scores(cand_hbm, state_hbm, dotp_hbm, sqp_hbm,
                   buf0, buf1, rd0, rd1, rs0, rs1, sv,
                   gs0, gs1, wd0, wd1, ws0, ws1):
        wid = lax.axis_index("s") * _NC + lax.axis_index("c")
        row0 = (_T + wid * _BPW) * _P
        pltpu.sync_copy(
            state_hbm.at[pl.ds((_T + wid * _BPW) * _D, _BPW * _D)], sv)

        bufs = (buf0, buf1)
        rds = (rd0, rd1)
        rss = (rs0, rs1)
        gsems = (gs0, gs1)
        wdsems = (wd0, wd1)
        wssems = (ws0, ws1)

        def start_gather(c):
            return pltpu.async_copy(
                cand_hbm.at[pl.ds(row0 + c * _RCH, _RCH)],
                bufs[c % 2], gsems[c % 2])

        def compute_chunk(c):
            k = c % 2
            buf, rd, rs = bufs[k], rds[k], rss[k]
            bi = c // _CPB

            def row_body(j, carry):
                r = j * 8
                for u in range(8):
                    v0 = buf[r + u, pl.ds(0, 16)]
                    dp = v0 * sv[pl.ds(bi * _D, 16)]
                    sp = v0 * v0
                    for kk in range(1, 8):
                        v = buf[r + u, pl.ds(kk * 16, 16)]
                        dp = dp + v * sv[pl.ds(bi * _D + kk * 16, 16)]
                        sp = sp + v * v
                    rd[pl.ds((r + u) * 16, 16)] = dp
                    rs[pl.ds((r + u) * 16, 16)] = sp
                return carry

            lax.fori_loop(0, _RCH // 8, row_body, 0)

        gcop = [None] * _NRCH
        wcop = [None] * _NRCH
        gcop[0] = start_gather(0)
        for c in range(_NRCH):
            k = c % 2
            if c + 1 < _NRCH:
                gcop[c + 1] = start_gather(c + 1)
            gcop[c].wait()
            if c >= 2:
                wcop[c - 2][0].wait()
                wcop[c - 2][1].wait()
            compute_chunk(c)
            off = (row0 - _T * _P + c * _RCH) * 16
            wcop[c] = (
                pltpu.async_copy(rds[k], dotp_hbm.at[pl.ds(off, _RCH * 16)],
                                 wdsems[k]),
                pltpu.async_copy(rss[k], sqp_hbm.at[pl.ds(off, _RCH * 16)],
                                 wssems[k]),
            )
        for c in (_NRCH - 2, _NRCH - 1):
            wcop[c][0].wait()
            wcop[c][1].wait()

    return _sc_scores


_FRB = 1024


def _finish_body(dp_ref, sp_ref, out_ref):
    seg = (lax.broadcasted_iota(jnp.int32, (_D, 8), 0) // 16
           == lax.broadcasted_iota(jnp.int32, (_D, 8), 1)).astype(jnp.float32)
    dn = (((1,), (0,)), ((), ()))
    dot = lax.dot_general(dp_ref[...], seg, dn,
                          preferred_element_type=jnp.float32)
    sq = lax.dot_general(sp_ref[...], seg, dn,
                         preferred_element_type=jnp.float32)
    out_ref[...] = dot / jnp.clip(jnp.sqrt(sq), 1e-12, None)


def kernel(state_input_ids, candidate_doc_embs, emb_table, W_enc, b_enc):
    ids = state_input_ids.reshape(_NW, _NCHUNK, _CHUNK).astype(jnp.int32)
    tok = _make_gather_rows()(ids, emb_table).reshape(_B, _L, _D)

    state = pl.pallas_call(
        _enc_body,
        grid=(_B // _BB,),
        in_specs=[
            pl.BlockSpec((_BB, _L, _D), lambda i: (i, 0, 0)),
            pl.BlockSpec((_D, _D), lambda i: (0, 0)),
            pl.BlockSpec((1, _D), lambda i: (0, 0)),
        ],
        out_specs=pl.BlockSpec((_BB, _D), lambda i: (i, 0)),
        out_shape=jax.ShapeDtypeStruct((_B, _D), jnp.float32),
    )(tok, W_enc, b_enc.reshape(1, _D))

    cand2 = candidate_doc_embs.reshape(_B * _P, _D)

    scores_tc = pl.pallas_call(
        _scores_body,
        grid=(_T // _NB,),
        in_specs=[
            pl.BlockSpec((_NB * _P, _D), lambda i: (i, 0)),
            pl.BlockSpec((1, _NB, _D), lambda i: (i, 0, 0)),
        ],
        out_specs=pl.BlockSpec((1, 1, _NB * _P), lambda i: (i, 0, 0)),
        out_shape=jax.ShapeDtypeStruct((_T // _NB, 1, _NB * _P), jnp.float32),
    )(cand2, state.reshape(_B // _NB, _NB, _D)[:_T // _NB])

    dotp, sqp = _make_sc_scores()(cand2, state.reshape(_B * _D))
    nfr = _SCB * _P // 8
    scores_sc = pl.pallas_call(
        _finish_body,
        grid=(nfr // _FRB,),
        in_specs=[
            pl.BlockSpec((_FRB, _D), lambda i: (i, 0)),
            pl.BlockSpec((_FRB, _D), lambda i: (i, 0)),
        ],
        out_specs=pl.BlockSpec((_FRB, 8), lambda i: (i, 0)),
        out_shape=jax.ShapeDtypeStruct((nfr, 8), jnp.float32),
    )(dotp.reshape(nfr, _D), sqp.reshape(nfr, _D))

    return jnp.concatenate(
        [scores_tc.reshape(_T, _P), scores_sc.reshape(_SCB, _P)], axis=0)


# speedup vs baseline: 1.2855x; 1.2855x over previous
import functools

import jax
import jax.numpy as jnp
from jax import lax
from jax.experimental import pallas as pl
from jax.experimental.pallas import tpu as pltpu
from jax.experimental.pallas import tpu_sc as plsc

_B, _L, _P, _D = 128, 200, 2048, 128
_BL = _B * _L

_NC, _NS = 2, 16
_NW = _NC * _NS

_PER_W = _BL // _NW
_CHUNK = 80
_NCHUNK = _PER_W // _CHUNK


@functools.cache
def _make_gather_rows():
    @functools.partial(
        pl.kernel,
        out_type=jax.ShapeDtypeStruct((_BL, _D), jnp.float32),
        mesh=plsc.VectorSubcoreMesh(core_axis_name="c", subcore_axis_name="s",
                                    num_cores=_NC, num_subcores=_NS),
        scratch_types=[
            pltpu.VMEM((_NCHUNK, _CHUNK), jnp.int32),
            pltpu.VMEM((_PER_W, _D), jnp.float32),
            pltpu.SemaphoreType.DMA,
        ],
    )
    def _gather_rows(idx_hbm, table_hbm, out_hbm, idx_v, rows_v, sem):
        wid = lax.axis_index("s") * _NC + lax.axis_index("c")
        base = wid * _PER_W
        pltpu.sync_copy(idx_hbm.at[wid], idx_v)
        copies = []
        for j in range(_NCHUNK):
            copies.append(pltpu.async_copy(
                table_hbm.at[idx_v.at[j]],
                rows_v.at[pl.ds(j * _CHUNK, _CHUNK)], sem))
        for c in copies:
            c.wait()
        pltpu.sync_copy(rows_v, out_hbm.at[pl.ds(base, _PER_W)])

    return _gather_rows


_BB = 16


def _enc_body(tok_ref, w_ref, b_ref, out_ref):
    t = tok_ref[...].reshape(_BB * _L, _D)
    y = jnp.tanh(jnp.dot(t, w_ref[...], preferred_element_type=jnp.float32)
                 + b_ref[...])
    m = jnp.mean(y.reshape(_BB, _L, _D), axis=1)
    n = jnp.sqrt(jnp.sum(m * m, axis=1, keepdims=True))
    out_ref[...] = m / jnp.clip(n, 1e-12, None)


_SCB = 32
_T = _B - _SCB
_BPW = _SCB // _NW
_RPW = _BPW * _P
_RCH = 256
_NRCH = _RPW // _RCH
_CPB = _P // _RCH

_DN = (((1,), (1,)), ((), ()))
_NB = 8


def _scores_body(cand_ref, state_ref, out_ref):
    s = state_ref[0].astype(jnp.bfloat16)
    ones = jnp.ones((1, _D), jnp.bfloat16)
    parts = []
    for b in range(_NB):
        c = cand_ref[b * _P:(b + 1) * _P]
        cb = c.astype(jnp.bfloat16)
        dot = lax.dot_general(s[b:b + 1], cb, _DN,
                              preferred_element_type=jnp.float32)
        sq = lax.dot_general(ones, cb * cb, _DN,
                             preferred_element_type=jnp.float32)
        parts.append(dot / jnp.clip(jnp.sqrt(sq), 1e-12, None))
    out_ref[0] = jnp.concatenate(parts, axis=1)


@functools.cache
def _make_sc_scores():
    @functools.partial(
        pl.kernel,
        out_type=(
            jax.ShapeDtypeStruct((_SCB * _P * 16,), jnp.float32),
            jax.ShapeDtypeStruct((_SCB * _P * 16,), jnp.float32),
        ),
        mesh=plsc.VectorSubcoreMesh(core_axis_name="c", subcore_axis_name="s",
                                    num_cores=_NC, num_subcores=_NS),
        scratch_types=[
            pltpu.VMEM((_RCH, _D), jnp.float32),
            pltpu.VMEM((_RCH, _D), jnp.float32),
            pltpu.VMEM((_RCH * 16,), jnp.float32),
            pltpu.VMEM((_RCH * 16,), jnp.float32),
            pltpu.VMEM((_RCH * 16,), jnp.float32),
            pltpu.VMEM((_RCH * 16,), jnp.float32),
            pltpu.VMEM((_BPW * _D,), jnp.float32),
            pltpu.SemaphoreType.DMA,
            pltpu.SemaphoreType.DMA,
            pltpu.SemaphoreType.DMA,
            pltpu.SemaphoreType.DMA,
            pltpu.SemaphoreType.DMA,
            pltpu.SemaphoreType.DMA,
        ],
    )
    def _sc_scores(cand_hbm, state_hbm, dotp_hbm, sqp_hbm,
                   buf0, buf1, rd0, rd1, rs0, rs1, sv,
                   gs0, gs1, wd0, wd1, ws0, ws1):
        wid = lax.axis_index("s") * _NC + lax.axis_index("c")
        row0 = (_T + wid * _BPW) * _P
        pltpu.sync_copy(
            state_hbm.at[pl.ds((_T + wid * _BPW) * _D, _BPW * _D)], sv)

        bufs = (buf0, buf1)
        rds = (rd0, rd1)
        rss = (rs0, rs1)
        gsems = (gs0, gs1)
        wdsems = (wd0, wd1)
        wssems = (ws0, ws1)

        def start_gather(c):
            return pltpu.async_copy(
                cand_hbm.at[pl.ds(row0 + c * _RCH, _RCH)],
                bufs[c % 2], gsems[c % 2])

        def compute_chunk(c):
            k = c % 2
            buf, rd, rs = bufs[k], rds[k], rss[k]
            bi = c // _CPB

            def row_body(j, carry):
                r = j * 8
                for u in range(8):
                    v0 = buf[r + u, pl.ds(0, 16)]
                    dp = v0 * sv[pl.ds(bi * _D, 16)]
                    sp = v0 * v0
                    for kk in range(1, 8):
                        v = buf[r + u, pl.ds(kk * 16, 16)]
                        dp = dp + v * sv[pl.ds(bi * _D + kk * 16, 16)]
                        sp = sp + v * v
                    rd[pl.ds((r + u) * 16, 16)] = dp
                    rs[pl.ds((r + u) * 16, 16)] = sp
                return carry

            lax.fori_loop(0, _RCH // 8, row_body, 0)

        gcop = [None] * _NRCH
        wcop = [None] * _NRCH
        gcop[0] = start_gather(0)
        for c in range(_NRCH):
            k = c % 2
            if c + 1 < _NRCH:
                gcop[c + 1] = start_gather(c + 1)
            gcop[c].wait()
            if c >= 2:
                wcop[c - 2][0].wait()
                wcop[c - 2][1].wait()
            compute_chunk(c)
            off = (row0 - _T * _P + c * _RCH) * 16
            wcop[c] = (
                pltpu.async_copy(rds[k], dotp_hbm.at[pl.ds(off, _RCH * 16)],
                                 wdsems[k]),
                pltpu.async_copy(rss[k], sqp_hbm.at[pl.ds(off, _RCH * 16)],
                                 wssems[k]),
            )
        for c in (_NRCH - 2, _NRCH - 1):
            wcop[c][0].wait()
            wcop[c][1].wait()

    return _sc_scores


_FRB = 1024


def _finish_body(dp_ref, sp_ref, out_ref):
    seg = (lax.broadcasted_iota(jnp.int32, (_D, 8), 0) // 16
           == lax.broadcasted_iota(jnp.int32, (_D, 8), 1)).astype(jnp.float32)
    dn = (((1,), (0,)), ((), ()))
    dot = lax.dot_general(dp_ref[...], seg, dn,
                          preferred_element_type=jnp.float32)
    sq = lax.dot_general(sp_ref[...], seg, dn,
                         preferred_element_type=jnp.float32)
    out_ref[...] = dot / jnp.clip(jnp.sqrt(sq), 1e-12, None)


def kernel(state_input_ids, candidate_doc_embs, emb_table, W_enc, b_enc):
    ids = state_input_ids.reshape(_NW, _NCHUNK, _CHUNK).astype(jnp.int32)
    tok = _make_gather_rows()(ids, emb_table).reshape(_B, _L, _D)

    state = pl.pallas_call(
        _enc_body,
        grid=(_B // _BB,),
        in_specs=[
            pl.BlockSpec((_BB, _L, _D), lambda i: (i, 0, 0)),
            pl.BlockSpec((_D, _D), lambda i: (0, 0)),
            pl.BlockSpec((1, _D), lambda i: (0, 0)),
        ],
        out_specs=pl.BlockSpec((_BB, _D), lambda i: (i, 0)),
        out_shape=jax.ShapeDtypeStruct((_B, _D), jnp.float32),
    )(tok, W_enc, b_enc.reshape(1, _D))

    cand2 = candidate_doc_embs.reshape(_B * _P, _D)

    scores_tc = pl.pallas_call(
        _scores_body,
        grid=(_T // _NB,),
        in_specs=[
            pl.BlockSpec((_NB * _P, _D), lambda i: (i, 0)),
            pl.BlockSpec((1, _NB, _D), lambda i: (i, 0, 0)),
        ],
        out_specs=pl.BlockSpec((1, 1, _NB * _P), lambda i: (i, 0, 0)),
        out_shape=jax.ShapeDtypeStruct((_T // _NB, 1, _NB * _P), jnp.float32),
    )(cand2, state.reshape(_B // _NB, _NB, _D)[:_T // _NB])

    dotp, sqp = _make_sc_scores()(cand2, state.reshape(_B * _D))
    nfr = _SCB * _P // 8
    scores_sc = pl.pallas_call(
        _finish_body,
        grid=(nfr // _FRB,),
        in_specs=[
            pl.BlockSpec((_FRB, _D), lambda i: (i, 0)),
            pl.BlockSpec((_FRB, _D), lambda i: (i, 0)),
        ],
        out_specs=pl.BlockSpec((_FRB, 8), lambda i: (i, 0)),
        out_shape=jax.ShapeDtypeStruct((nfr, 8), jnp.float32),
    )(dotp.reshape(nfr, _D), sqp.reshape(nfr, _D))

    return jnp.concatenate(
        [scores_tc.reshape(_T, _P), scores_sc.reshape(_SCB, _P)], axis=0)


# speedup vs baseline: 1.4670x; 1.1412x over previous
import functools

import jax
import jax.numpy as jnp
from jax import lax
from jax.experimental import pallas as pl
from jax.experimental.pallas import tpu as pltpu
from jax.experimental.pallas import tpu_sc as plsc

_B, _L, _P, _D = 128, 200, 2048, 128
_BL = _B * _L

_NC, _NS = 2, 16
_NW = _NC * _NS
_PER_W = _BL // _NW
_CHUNK = 80
_NCHUNK = _PER_W // _CHUNK


@functools.cache
def _make_gather_rows():
    @functools.partial(
        pl.kernel,
        out_type=jax.ShapeDtypeStruct((_BL, _D), jnp.float32),
        mesh=plsc.VectorSubcoreMesh(core_axis_name="c", subcore_axis_name="s",
                                    num_cores=_NC, num_subcores=_NS),
        scratch_types=[
            pltpu.VMEM((_NCHUNK, _CHUNK), jnp.int32),
            pltpu.VMEM((_PER_W, _D), jnp.float32),
            pltpu.SemaphoreType.DMA,
        ],
    )
    def _gather_rows(idx_hbm, table_hbm, out_hbm, idx_v, rows_v, sem):
        wid = lax.axis_index("s") * _NC + lax.axis_index("c")
        base = wid * _PER_W
        pltpu.sync_copy(idx_hbm.at[wid], idx_v)
        copies = []
        for j in range(_NCHUNK):
            copies.append(pltpu.async_copy(
                table_hbm.at[idx_v.at[j]],
                rows_v.at[pl.ds(j * _CHUNK, _CHUNK)], sem))
        for c in copies:
            c.wait()
        pltpu.sync_copy(rows_v, out_hbm.at[pl.ds(base, _PER_W)])

    return _gather_rows


_BB = 16


def _enc_body(tok_ref, w_ref, b_ref, out_ref):
    t = tok_ref[...].reshape(_BB * _L, _D)
    y = jnp.tanh(jnp.dot(t, w_ref[...], preferred_element_type=jnp.float32)
                 + b_ref[...])
    m = jnp.mean(y.reshape(_BB, _L, _D), axis=1)
    n = jnp.sqrt(jnp.sum(m * m, axis=1, keepdims=True))
    out_ref[...] = m / jnp.clip(n, 1e-12, None)


_DN = (((1,), (1,)), ((), ()))
_NB = 8


def _scores_body(cand_ref, state_ref, out_ref):
    s = state_ref[0].astype(jnp.bfloat16)
    ones = jnp.ones((1, _D), jnp.bfloat16)
    parts = []
    for b in range(_NB):
        c = cand_ref[b * _P:(b + 1) * _P]
        cb = c.astype(jnp.bfloat16)
        dot = lax.dot_general(s[b:b + 1], cb, _DN,
                              preferred_element_type=jnp.float32)
        sq = lax.dot_general(ones, cb * cb, _DN,
                             preferred_element_type=jnp.float32)
        parts.append(dot / jnp.clip(jnp.sqrt(sq), 1e-12, None))
    out_ref[0] = jnp.concatenate(parts, axis=1)


def kernel(state_input_ids, candidate_doc_embs, emb_table, W_enc, b_enc):
    ids = state_input_ids.reshape(_NW, _NCHUNK, _CHUNK).astype(jnp.int32)
    tok = _make_gather_rows()(ids, emb_table).reshape(_B, _L, _D)

    state = pl.pallas_call(
        _enc_body,
        grid=(_B // _BB,),
        in_specs=[
            pl.BlockSpec((_BB, _L, _D), lambda i: (i, 0, 0)),
            pl.BlockSpec((_D, _D), lambda i: (0, 0)),
            pl.BlockSpec((1, _D), lambda i: (0, 0)),
        ],
        out_specs=pl.BlockSpec((_BB, _D), lambda i: (i, 0)),
        out_shape=jax.ShapeDtypeStruct((_B, _D), jnp.float32),
    )(tok, W_enc, b_enc.reshape(1, _D))

    cand2 = candidate_doc_embs.reshape(_B * _P, _D)
    scores = pl.pallas_call(
        _scores_body,
        grid=(_B // _NB,),
        in_specs=[
            pl.BlockSpec((_NB * _P, _D), lambda i: (i, 0)),
            pl.BlockSpec((1, _NB, _D), lambda i: (i, 0, 0)),
        ],
        out_specs=pl.BlockSpec((1, 1, _NB * _P), lambda i: (i, 0, 0)),
        out_shape=jax.ShapeDtypeStruct((_B // _NB, 1, _NB * _P), jnp.float32),
    )(cand2, state.reshape(_B // _NB, _NB, _D))
    return scores.reshape(_B, _P)
